# baseline (speedup 1.0000x reference)
import jax
import jax.numpy as jnp
from jax.experimental import pallas as pl
from jax.experimental.pallas import tpu as pltpu

_D = 128
_BLOCK = 2504


def _sigmoid(z):
    return 0.5 + 0.5 * jnp.tanh(0.5 * z)


def _gclstm_body(x_ref, wi_ref, wc_ref, wo_ref, bchi_ref, bchc_ref, bcho_ref,
                 bi_ref, bc_ref, bo_ref, wco_ref, fcw_ref, fcb_ref, o_ref):
    x = x_ref[...]
    xi = jnp.dot(x, wi_ref[...], preferred_element_type=jnp.float32)
    xc = jnp.dot(x, wc_ref[...], preferred_element_type=jnp.float32)
    xo = jnp.dot(x, wo_ref[...], preferred_element_type=jnp.float32)
    gi = _sigmoid(xi + (bchi_ref[...] + bi_ref[...]))
    gt = jnp.tanh(xc + (bchc_ref[...] + bc_ref[...]))
    cn = gi * gt
    go = _sigmoid(xo + (bcho_ref[...] + bo_ref[...]) + wco_ref[...] * cn)
    hn = go * jnp.tanh(cn)
    out = jax.lax.dot_general(hn, fcw_ref[...],
                              dimension_numbers=(((1,), (1,)), ((), ())),
                              preferred_element_type=jnp.float32)
    o_ref[...] = out + fcb_ref[...]


def kernel(x, edge_index, batch, W_i, W_f, W_c, W_o, Th_i, Th_f, Th_c, Th_o,
           bch_i, bch_f, bch_c, bch_o, w_ci, w_cf, w_co, b_i, b_f, b_c, b_o,
           fc_w, fc_b):
    n = x.shape[0]
    full = lambda shape: pl.BlockSpec(shape, lambda i: (0,) * len(shape))
    return pl.pallas_call(
        _gclstm_body,
        grid=(pl.cdiv(n, _BLOCK),),
        in_specs=[
            pl.BlockSpec((_BLOCK, _D), lambda i: (i, 0)),
            full((_D, _D)), full((_D, _D)), full((_D, _D)),
            full((1, _D)), full((1, _D)), full((1, _D)),
            full((1, _D)), full((1, _D)), full((1, _D)),
            full((1, _D)), full((_D, _D)), full((1, _D)),
        ],
        out_specs=pl.BlockSpec((_BLOCK, _D), lambda i: (i, 0)),
        out_shape=jax.ShapeDtypeStruct((n, _D), jnp.float32),
        compiler_params=pltpu.CompilerParams(
            dimension_semantics=("parallel",)),
    )(x, W_i, W_c, W_o, bch_i[None, :], bch_c[None, :], bch_o[None, :],
      b_i, b_c, b_o, w_co, fc_w, fc_b[None, :])


# speedup vs baseline: 1.1410x; 1.1410x over previous
import jax
import jax.numpy as jnp
from jax.experimental import pallas as pl
from jax.experimental.pallas import tpu as pltpu

_D = 128
_BLOCK = 10000


def _sigmoid(z):
    return 0.5 + 0.5 * jnp.tanh(0.5 * z)


def _gclstm_body(x_ref, wi_ref, wc_ref, wo_ref, bchi_ref, bchc_ref, bcho_ref,
                 bi_ref, bc_ref, bo_ref, wco_ref, fcw_ref, fcb_ref, o_ref):
    x = x_ref[...]
    xi = jnp.dot(x, wi_ref[...], preferred_element_type=jnp.float32)
    xc = jnp.dot(x, wc_ref[...], preferred_element_type=jnp.float32)
    xo = jnp.dot(x, wo_ref[...], preferred_element_type=jnp.float32)
    gi = _sigmoid(xi + (bchi_ref[...] + bi_ref[...]))
    gt = jnp.tanh(xc + (bchc_ref[...] + bc_ref[...]))
    cn = gi * gt
    go = _sigmoid(xo + (bcho_ref[...] + bo_ref[...]) + wco_ref[...] * cn)
    hn = go * jnp.tanh(cn)
    out = jax.lax.dot_general(hn, fcw_ref[...],
                              dimension_numbers=(((1,), (1,)), ((), ())),
                              preferred_element_type=jnp.float32)
    o_ref[...] = out + fcb_ref[...]


def kernel(x, edge_index, batch, W_i, W_f, W_c, W_o, Th_i, Th_f, Th_c, Th_o,
           bch_i, bch_f, bch_c, bch_o, w_ci, w_cf, w_co, b_i, b_f, b_c, b_o,
           fc_w, fc_b):
    n = x.shape[0]
    full = lambda shape: pl.BlockSpec(shape, lambda i: (0,) * len(shape))
    return pl.pallas_call(
        _gclstm_body,
        grid=(pl.cdiv(n, _BLOCK),),
        in_specs=[
            pl.BlockSpec((_BLOCK, _D), lambda i: (i, 0)),
            full((_D, _D)), full((_D, _D)), full((_D, _D)),
            full((1, _D)), full((1, _D)), full((1, _D)),
            full((1, _D)), full((1, _D)), full((1, _D)),
            full((1, _D)), full((_D, _D)), full((1, _D)),
        ],
        out_specs=pl.BlockSpec((_BLOCK, _D), lambda i: (i, 0)),
        out_shape=jax.ShapeDtypeStruct((n, _D), jnp.float32),
        compiler_params=pltpu.CompilerParams(
            dimension_semantics=("parallel",)),
    )(x, W_i, W_c, W_o, bch_i[None, :], bch_c[None, :], bch_o[None, :],
      b_i, b_c, b_o, w_co, fc_w, fc_b[None, :])
